# baseline (speedup 1.0000x reference)
import functools

import jax
import jax.numpy as jnp
from jax import lax
from jax.experimental import pallas as pl
from jax.experimental.pallas import tpu as pltpu
from jax.experimental.pallas import tpu_sc as plsc

D_MODEL = 64
SCALE = 8.0
C = 128


def kernel(x, table):
    out_shape = (*x.shape, D_MODEL)
    B = x.size
    V = table.shape[0]

    info = plsc.get_sparse_core_info()
    NC, NS = info.num_cores, info.num_subcores
    NW = NC * NS
    BPW = B // NW
    NCH = BPW // C
    assert BPW * NW == B and NCH * C == BPW and NCH % 2 == 0

    x_rows = jnp.reshape(x.astype(jnp.int32), (NW * NCH, C))
    table2 = jnp.reshape(table, (V // 2, 2 * D_MODEL))

    mesh = plsc.VectorSubcoreMesh(core_axis_name="c", subcore_axis_name="s")

    @functools.partial(
        pl.kernel,
        mesh=mesh,
        out_type=jax.ShapeDtypeStruct((B, D_MODEL), jnp.float32),
        scratch_types=[
            pltpu.VMEM((NCH, C), jnp.int32),
            pltpu.VMEM((2, C), jnp.int32),
            pltpu.VMEM((2, C, 2 * D_MODEL), jnp.float32),
            pltpu.VMEM((C, D_MODEL), jnp.float32),
            pltpu.SemaphoreType.DMA,
            pltpu.SemaphoreType.DMA,
        ],
    )
    def emb(x_hbm, table_hbm, out_hbm, idx_all, kbuf, rows, outb, sem0, sem1):
        wid = lax.axis_index("c") * NS + lax.axis_index("s")
        pltpu.sync_copy(x_hbm.at[pl.ds(wid * NCH, NCH)], idx_all)

        sems = (sem0, sem1)

        def prep(n, b):
            def body(m, _):
                sl = pl.ds(m * 16, 16)
                kbuf[b, sl] = lax.shift_right_logical(idx_all[n, sl], 1)
                return 0
            lax.fori_loop(0, C // 16, body, 0)

        def gather_start(b):
            pltpu.make_async_copy(
                table_hbm.at[kbuf.at[b]], rows.at[b], sems[b]
            ).start()

        def gather_wait(b):
            pltpu.make_async_copy(
                table_hbm.at[kbuf.at[b]], rows.at[b], sems[b]
            ).wait()

        prep(0, 0)
        gather_start(0)

        out_base = wid * BPW

        def outer(i, _):
            n0 = i * 2
            for b in range(2):
                n = n0 + b
                nxt = n + 1

                @pl.when(nxt < NCH)
                def _():
                    prep(nxt, 1 - b)
                    gather_start(1 - b)

                gather_wait(b)

                def scale(g, _):
                    hv = lax.bitwise_and(idx_all[n, pl.ds(g * 16, 16)], 1)
                    off = hv * D_MODEL
                    for rr in range(16):
                        r = g * 16 + rr
                        o = off[rr]
                        for m in range(D_MODEL // 16):
                            outb[r, pl.ds(m * 16, 16)] = (
                                rows[b, r, pl.ds(o + m * 16, 16)] * SCALE
                            )
                    return 0

                lax.fori_loop(0, C // 16, scale, 0)

                pltpu.sync_copy(outb, out_hbm.at[pl.ds(out_base + n * C, C)])
            return 0

        lax.fori_loop(0, NCH // 2, outer, 0)

    out = emb(x_rows, table2)
    return out.reshape(out_shape)


# speedup vs baseline: 1.0162x; 1.0162x over previous
import functools

import jax
import jax.numpy as jnp
from jax import lax
from jax.experimental import pallas as pl
from jax.experimental.pallas import tpu as pltpu
from jax.experimental.pallas import tpu_sc as plsc

D_MODEL = 64
SCALE = 8.0
C = 128


def kernel(x, table):
    NB, NT = x.shape
    B = NB * NT
    V = table.shape[0]

    info = plsc.get_sparse_core_info()
    NC, NS = info.num_cores, info.num_subcores
    NW = NC * NS
    BPW = B // NW
    NCH = BPW // C
    XR = BPW // NT
    assert BPW * NW == B and NCH * C == BPW and NCH % 2 == 0
    assert XR * NT == BPW and NT % 8 == 0

    xi = jnp.reshape(x.astype(jnp.int32), (NW * NCH, C))
    table2 = jnp.reshape(table, (V // 2, 2 * D_MODEL))

    mesh = plsc.VectorSubcoreMesh(core_axis_name="c", subcore_axis_name="s")

    @functools.partial(
        pl.kernel,
        mesh=mesh,
        out_type=jax.ShapeDtypeStruct((NB, NT, D_MODEL), jnp.float32),
        scratch_types=[
            pltpu.VMEM((NCH, C), jnp.int32),
            pltpu.VMEM((2, C), jnp.int32),
            pltpu.VMEM((2, C, 2 * D_MODEL), jnp.float32),
            pltpu.VMEM((2, C, D_MODEL), jnp.float32),
            pltpu.SemaphoreType.DMA,
            pltpu.SemaphoreType.DMA,
            pltpu.SemaphoreType.DMA,
            pltpu.SemaphoreType.DMA,
        ],
    )
    def emb(x_hbm, table_hbm, out_hbm, idx_all, kbuf, pairs, outb,
            gsem0, gsem1, osem0, osem1):
        wid = lax.axis_index("c") * NS + lax.axis_index("s")
        oview = out_hbm.reshape(B, D_MODEL)

        pltpu.sync_copy(x_hbm.at[pl.ds(wid * NCH, NCH)], idx_all)

        gsems = (gsem0, gsem1)
        osems = (osem0, osem1)

        def prep(n, b):
            def body(m, _):
                sl = pl.ds(m * 16, 16)
                kbuf[b, sl] = lax.shift_right_logical(idx_all[n, sl], 1)
                return 0
            lax.fori_loop(0, C // 16, body, 0)

        def gather_start(b):
            pltpu.make_async_copy(
                table_hbm.at[kbuf.at[b]], pairs.at[b], gsems[b]
            ).start()

        def gather_wait(b):
            pltpu.make_async_copy(
                table_hbm.at[kbuf.at[b]], pairs.at[b], gsems[b]
            ).wait()

        def store_start(n, b):
            pltpu.make_async_copy(
                outb.at[b], oview.at[pl.ds(wid * BPW + n * C, C)], osems[b]
            ).start()

        def store_wait(n, b):
            pltpu.make_async_copy(
                outb.at[b], oview.at[pl.ds(wid * BPW + n * C, C)], osems[b]
            ).wait()

        prep(0, 0)
        gather_start(0)

        def outer(i, _):
            n0 = i * 2
            for b in range(2):
                n = n0 + b
                nxt = n + 1

                @pl.when(nxt < NCH)
                def _():
                    prep(nxt, 1 - b)
                    gather_start(1 - b)

                gather_wait(b)

                @pl.when(n >= 2)
                def _():
                    store_wait(n - 2, b)

                def scale(g, _):
                    hv = lax.bitwise_and(idx_all[n, pl.ds(g * 16, 16)], 1)
                    off = hv * D_MODEL
                    for rr in range(16):
                        r = g * 16 + rr
                        o = off[rr]
                        for m in range(D_MODEL // 16):
                            outb[b, r, pl.ds(m * 16, 16)] = (
                                pairs[b, r, pl.ds(o + m * 16, 16)] * SCALE
                            )
                    return 0

                lax.fori_loop(0, C // 16, scale, 0)

                store_start(n, b)
            return 0

        lax.fori_loop(0, NCH // 2, outer, 0)
        store_wait(NCH - 2, 0)
        store_wait(NCH - 1, 1)

    return emb(xi, table2)
